# Initial kernel scaffold; baseline (speedup 1.0000x reference)
import jax
import jax.numpy as jnp
from jax.experimental import pallas as pl


def kernel(x, edge_index, Wi, bi, Wl, bl, Wr, gamma, beta, Wo, bo):
    raise NotImplementedError("write your pallas kernel here")


# speedup vs baseline: 4.0833x; 4.0833x over previous
import functools

import jax
import jax.numpy as jnp
from jax import lax
from jax.experimental import pallas as pl
from jax.experimental.pallas import tpu as pltpu
from jax.experimental.pallas import tpu_sc as plsc

N = 10000
E = 320000
D = 128
H = 128
OUT = 2
LAYERS = 3

NC = 2
NS = 16
NW = NC * NS
EPW = E // NW
CH = 80
NCH = EPW // CH
ZCH = 200
NZ = N // ZCH
ZPT = -(-NZ // NS)

_mesh = plsc.VectorSubcoreMesh(
    core_axis_name="c", subcore_axis_name="s", num_cores=NC, num_subcores=NS
)


@functools.partial(
    pl.kernel,
    out_type=jax.ShapeDtypeStruct((NC, N, H), jnp.float32),
    mesh=_mesh,
    scratch_types=[
        pltpu.VMEM((CH,), jnp.int32),
        pltpu.VMEM((CH,), jnp.int32),
        pltpu.VMEM((CH, H), jnp.float32),
        pltpu.VMEM((ZCH, H), jnp.float32),
        pltpu.VMEM_SHARED((N, H), jnp.float32),
        pltpu.SemaphoreType.DMA,
    ],
)
def _sc_agg(src_hbm, dst_hbm, h_hbm, out_hbm, src_v, dst_v, rows_v, zer_v,
            agg_sh, sem):
    c = lax.axis_index("c")
    s = lax.axis_index("s")
    wid = c * NS + s

    def zbody(i, _):
        for k in range(H // 16):
            zer_v[i, pl.ds(k * 16, 16)] = jnp.zeros((16,), jnp.float32)
        return 0

    lax.fori_loop(0, ZCH, zbody, 0)

    for j in range(ZPT):
        cid = s + j * NS

        @pl.when(cid < NZ)
        def _():
            pltpu.sync_copy(
                zer_v, agg_sh.at[pl.ds(pl.multiple_of(cid * ZCH, ZCH), ZCH)]
            )

    plsc.subcore_barrier()

    def ebody(g, _):
        base = pl.multiple_of(wid * EPW + g * CH, CH)
        pltpu.sync_copy(src_hbm.at[pl.ds(base, CH)], src_v)
        pltpu.sync_copy(dst_hbm.at[pl.ds(base, CH)], dst_v)
        pltpu.async_copy(h_hbm.at[src_v], rows_v, sem).wait()
        pltpu.sync_copy(rows_v, agg_sh.at[dst_v], add=True)
        return 0

    lax.fori_loop(0, NCH, ebody, 0)
    plsc.subcore_barrier()

    for j in range(ZPT):
        cid = s + j * NS

        @pl.when(cid < NZ)
        def _():
            r0 = pl.multiple_of(cid * ZCH, ZCH)
            pltpu.sync_copy(agg_sh.at[pl.ds(r0, ZCH)], zer_v)
            pltpu.sync_copy(zer_v, out_hbm.at[c, pl.ds(r0, ZCH)])


RB = 1000


def _tc_init_body(x_ref, wi_ref, bi_ref, o_ref):
    o_ref[...] = jnp.maximum(
        jnp.dot(x_ref[...], wi_ref[...], preferred_element_type=jnp.float32)
        + bi_ref[...],
        0.0,
    )


_tc_init = pl.pallas_call(
    _tc_init_body,
    grid=(N // RB,),
    in_specs=[
        pl.BlockSpec((RB, D), lambda i: (i, 0)),
        pl.BlockSpec((D, H), lambda i: (0, 0)),
        pl.BlockSpec((1, H), lambda i: (0, 0)),
    ],
    out_specs=pl.BlockSpec((RB, H), lambda i: (i, 0)),
    out_shape=jax.ShapeDtypeStruct((N, H), jnp.float32),
)


def _tc_layer_body(agg_ref, deg_ref, h_ref, wl_ref, bl_ref, wr_ref, g_ref,
                   b_ref, o_ref):
    d = jnp.clip(deg_ref[0] + deg_ref[1], 1.0, None)
    a = (agg_ref[0] + agg_ref[1]) / d
    h = h_ref[...]
    h2 = (
        jnp.dot(a, wl_ref[...], preferred_element_type=jnp.float32)
        + bl_ref[...]
        + jnp.dot(h, wr_ref[...], preferred_element_type=jnp.float32)
    )
    mu = jnp.mean(h2, axis=-1, keepdims=True)
    var = jnp.mean((h2 - mu) ** 2, axis=-1, keepdims=True)
    h2 = (h2 - mu) * lax.rsqrt(var + 1e-5) * g_ref[...] + b_ref[...]
    o_ref[...] = jnp.maximum(h2, 0.0) + h


_tc_layer = pl.pallas_call(
    _tc_layer_body,
    grid=(N // RB,),
    in_specs=[
        pl.BlockSpec((NC, RB, H), lambda i: (0, i, 0)),
        pl.BlockSpec((NC, RB, 1), lambda i: (0, i, 0)),
        pl.BlockSpec((RB, H), lambda i: (i, 0)),
        pl.BlockSpec((H, H), lambda i: (0, 0)),
        pl.BlockSpec((1, H), lambda i: (0, 0)),
        pl.BlockSpec((H, H), lambda i: (0, 0)),
        pl.BlockSpec((1, H), lambda i: (0, 0)),
        pl.BlockSpec((1, H), lambda i: (0, 0)),
    ],
    out_specs=pl.BlockSpec((RB, H), lambda i: (i, 0)),
    out_shape=jax.ShapeDtypeStruct((N, H), jnp.float32),
)


def _tc_final_body(h_ref, wo_ref, bo_ref, o_ref):
    logits = (
        jnp.dot(h_ref[...], wo_ref[...], preferred_element_type=jnp.float32)
        + bo_ref[...]
    )
    m = jnp.max(logits, axis=-1, keepdims=True)
    lse = jnp.log(jnp.sum(jnp.exp(logits - m), axis=-1, keepdims=True)) + m
    o_ref[...] = logits - lse


_tc_final = pl.pallas_call(
    _tc_final_body,
    grid=(N // RB,),
    in_specs=[
        pl.BlockSpec((RB, H), lambda i: (i, 0)),
        pl.BlockSpec((H, OUT), lambda i: (0, 0)),
        pl.BlockSpec((1, OUT), lambda i: (0, 0)),
    ],
    out_specs=pl.BlockSpec((RB, OUT), lambda i: (i, 0)),
    out_shape=jax.ShapeDtypeStruct((N, OUT), jnp.float32),
)


def kernel(x, edge_index, Wi, bi, Wl, bl, Wr, gamma, beta, Wo, bo):
    src = edge_index[0]
    dst = edge_index[1]
    deg2 = _sc_agg(src, dst, jnp.ones((N, H), jnp.float32))[:, :, :1]
    h = _tc_init(x, Wi, bi[None, :])
    for i in range(LAYERS):
        agg2 = _sc_agg(src, dst, h)
        h = _tc_layer(
            agg2, deg2, h, Wl[i], bl[i][None, :], Wr[i], gamma[i][None, :],
            beta[i][None, :],
        )
    return _tc_final(h, Wo, bo[None, :])


# speedup vs baseline: 11.8778x; 2.9089x over previous
import functools

import jax
import jax.numpy as jnp
from jax import lax
from jax.experimental import pallas as pl
from jax.experimental.pallas import tpu as pltpu
from jax.experimental.pallas import tpu_sc as plsc

N = 10000
E = 320000
D = 128
H = 128
OUT = 2
LAYERS = 3

NC = 2
NS = 16
NW = NC * NS
EPW = E // NW
CH = 125
NCH = EPW // CH
IB = 16
NB = NCH // IB
ZCH = 80
NZ = N // ZCH
ZPT = -(-NZ // NS)

_mesh = plsc.VectorSubcoreMesh(
    core_axis_name="c", subcore_axis_name="s", num_cores=NC, num_subcores=NS
)


def _make_sc_agg(W):
    @functools.partial(
        pl.kernel,
        out_type=jax.ShapeDtypeStruct((NC, N, W), jnp.float32),
        mesh=_mesh,
        scratch_types=[
            pltpu.VMEM((IB, CH), jnp.int32),
            pltpu.VMEM((IB, CH), jnp.int32),
            pltpu.VMEM((CH, W), jnp.float32),
            pltpu.VMEM((CH, W), jnp.float32),
            pltpu.VMEM_SHARED((N, W), jnp.float32),
            pltpu.SemaphoreType.DMA,
            pltpu.SemaphoreType.DMA,
        ],
    )
    def sc_agg(src_hbm, dst_hbm, h_hbm, out_hbm, srcs_v, dsts_v, rows0, rows1,
               agg_sh, sem0, sem1):
        c = lax.axis_index("c")
        s = lax.axis_index("s")
        wid = c * NS + s
        rows = (rows0, rows1)
        sems = (sem0, sem1)

        def zbody(i, _):
            for k in range(W // 16):
                rows0[i, pl.ds(k * 16, 16)] = jnp.zeros((16,), jnp.float32)
            return 0

        lax.fori_loop(0, ZCH, zbody, 0)

        for j in range(ZPT):
            cid = s + j * NS

            @pl.when(cid < NZ)
            def _():
                pltpu.sync_copy(
                    rows0.at[pl.ds(0, ZCH)],
                    agg_sh.at[pl.ds(pl.multiple_of(cid * ZCH, ZCH), ZCH)],
                )

        plsc.subcore_barrier()

        def bbody(blk, _):
            b0 = pl.multiple_of(blk * IB, IB)
            pltpu.sync_copy(src_hbm.at[wid, pl.ds(b0, IB)], srcs_v)
            pltpu.sync_copy(dst_hbm.at[wid, pl.ds(b0, IB)], dsts_v)
            pltpu.async_copy(h_hbm.at[srcs_v.at[0]], rows0, sem0)
            pltpu.async_copy(h_hbm.at[srcs_v.at[1]], rows1, sem1)

            def ebody(g2, _):
                for b in range(2):
                    g = g2 * 2 + b
                    pltpu.make_async_copy(
                        h_hbm.at[srcs_v.at[g]], rows[b], sems[b]
                    ).wait()
                    pltpu.sync_copy(rows[b], agg_sh.at[dsts_v.at[g]], add=True)

                    @pl.when(g + 2 < IB)
                    def _():
                        pltpu.async_copy(
                            h_hbm.at[srcs_v.at[g + 2]], rows[b], sems[b]
                        )

                return 0

            lax.fori_loop(0, IB // 2, ebody, 0)
            return 0

        lax.fori_loop(0, NB, bbody, 0)
        plsc.subcore_barrier()

        for j in range(ZPT):
            cid = s + j * NS

            @pl.when(cid < NZ)
            def _():
                r0 = pl.multiple_of(cid * ZCH, ZCH)
                pltpu.sync_copy(agg_sh.at[pl.ds(r0, ZCH)], rows0.at[pl.ds(0, ZCH)])
                pltpu.sync_copy(rows0.at[pl.ds(0, ZCH)], out_hbm.at[c, pl.ds(r0, ZCH)])

    return sc_agg


_sc_agg = _make_sc_agg(H)


@functools.partial(
    pl.kernel,
    out_type=jax.ShapeDtypeStruct((NC, N), jnp.float32),
    mesh=_mesh,
    scratch_types=[
        pltpu.VMEM((NCH, CH), jnp.int32),
        pltpu.VMEM((CH,), jnp.float32),
        pltpu.VMEM((N,), jnp.float32),
        pltpu.VMEM_SHARED((N,), jnp.float32),
    ],
)
def _sc_deg(dst_hbm, out_hbm, dsts_v, ones_v, degw_v, deg_sh):
    c = lax.axis_index("c")
    s = lax.axis_index("s")
    wid = c * NS + s

    pltpu.sync_copy(dst_hbm.at[wid], dsts_v)

    for r0 in list(range(0, CH - 16, 16)) + [CH - 16]:
        ones_v[pl.ds(r0, 16)] = jnp.full((16,), 1.0, jnp.float32)

    def zbody(i, _):
        degw_v[pl.ds(i * 16, 16)] = jnp.zeros((16,), jnp.float32)
        return 0

    lax.fori_loop(0, N // 16, zbody, 0)

    @pl.when(s == 0)
    def _():
        pltpu.sync_copy(degw_v, deg_sh)

    plsc.subcore_barrier()

    def ebody(g, _):
        pltpu.sync_copy(ones_v, deg_sh.at[dsts_v.at[g]], add=True)
        return 0

    lax.fori_loop(0, NCH, ebody, 0)
    plsc.subcore_barrier()

    @pl.when(s == 0)
    def _():
        pltpu.sync_copy(deg_sh, degw_v)
        pltpu.sync_copy(degw_v, out_hbm.at[c])


RB = 1000


def _tc_init_body(x_ref, wi_ref, bi_ref, o_ref):
    o_ref[...] = jnp.maximum(
        jnp.dot(x_ref[...], wi_ref[...], preferred_element_type=jnp.float32)
        + bi_ref[...],
        0.0,
    )


_tc_init = pl.pallas_call(
    _tc_init_body,
    grid=(N // RB,),
    in_specs=[
        pl.BlockSpec((RB, D), lambda i: (i, 0)),
        pl.BlockSpec((D, H), lambda i: (0, 0)),
        pl.BlockSpec((1, H), lambda i: (0, 0)),
    ],
    out_specs=pl.BlockSpec((RB, H), lambda i: (i, 0)),
    out_shape=jax.ShapeDtypeStruct((N, H), jnp.float32),
)


def _tc_layer_body(agg_ref, deg_ref, h_ref, wl_ref, bl_ref, wr_ref, g_ref,
                   b_ref, o_ref):
    d = jnp.clip(deg_ref[0] + deg_ref[1], 1.0, None)
    a = (agg_ref[0] + agg_ref[1]) / d
    h = h_ref[...]
    h2 = (
        jnp.dot(a, wl_ref[...], preferred_element_type=jnp.float32)
        + bl_ref[...]
        + jnp.dot(h, wr_ref[...], preferred_element_type=jnp.float32)
    )
    mu = jnp.mean(h2, axis=-1, keepdims=True)
    var = jnp.mean((h2 - mu) ** 2, axis=-1, keepdims=True)
    h2 = (h2 - mu) * lax.rsqrt(var + 1e-5) * g_ref[...] + b_ref[...]
    o_ref[...] = jnp.maximum(h2, 0.0) + h


_tc_layer = pl.pallas_call(
    _tc_layer_body,
    grid=(N // RB,),
    in_specs=[
        pl.BlockSpec((NC, RB, H), lambda i: (0, i, 0)),
        pl.BlockSpec((NC, RB, 1), lambda i: (0, i, 0)),
        pl.BlockSpec((RB, H), lambda i: (i, 0)),
        pl.BlockSpec((H, H), lambda i: (0, 0)),
        pl.BlockSpec((1, H), lambda i: (0, 0)),
        pl.BlockSpec((H, H), lambda i: (0, 0)),
        pl.BlockSpec((1, H), lambda i: (0, 0)),
        pl.BlockSpec((1, H), lambda i: (0, 0)),
    ],
    out_specs=pl.BlockSpec((RB, H), lambda i: (i, 0)),
    out_shape=jax.ShapeDtypeStruct((N, H), jnp.float32),
)


def _tc_final_body(h_ref, wo_ref, bo_ref, o_ref):
    logits = (
        jnp.dot(h_ref[...], wo_ref[...], preferred_element_type=jnp.float32)
        + bo_ref[...]
    )
    m = jnp.max(logits, axis=-1, keepdims=True)
    lse = jnp.log(jnp.sum(jnp.exp(logits - m), axis=-1, keepdims=True)) + m
    o_ref[...] = logits - lse


_tc_final = pl.pallas_call(
    _tc_final_body,
    grid=(N // RB,),
    in_specs=[
        pl.BlockSpec((RB, H), lambda i: (i, 0)),
        pl.BlockSpec((H, OUT), lambda i: (0, 0)),
        pl.BlockSpec((1, OUT), lambda i: (0, 0)),
    ],
    out_specs=pl.BlockSpec((RB, OUT), lambda i: (i, 0)),
    out_shape=jax.ShapeDtypeStruct((N, OUT), jnp.float32),
)


def kernel(x, edge_index, Wi, bi, Wl, bl, Wr, gamma, beta, Wo, bo):
    src = edge_index[0].reshape(NW, NCH, CH)
    dst = edge_index[1].reshape(NW, NCH, CH)
    deg2 = _sc_deg(dst)[:, :, None]
    h = _tc_init(x, Wi, bi[None, :])
    for i in range(LAYERS):
        agg2 = _sc_agg(src, dst, h)
        h = _tc_layer(
            agg2, deg2, h, Wl[i], bl[i][None, :], Wr[i], gamma[i][None, :],
            beta[i][None, :],
        )
    return _tc_final(h, Wo, bo[None, :])


# speedup vs baseline: 11.9171x; 1.0033x over previous
import functools

import jax
import jax.numpy as jnp
from jax import lax
from jax.experimental import pallas as pl
from jax.experimental.pallas import tpu as pltpu
from jax.experimental.pallas import tpu_sc as plsc

N = 10000
E = 320000
D = 128
H = 128
OUT = 2
LAYERS = 3

NC = 2
NS = 16
NW = NC * NS
EPW = E // NW
CH = 125
NCH = EPW // CH
IB = 16
NB = NCH // IB
ZCH = 80
NZ = N // ZCH
ZPT = -(-NZ // NS)

_mesh = plsc.VectorSubcoreMesh(
    core_axis_name="c", subcore_axis_name="s", num_cores=NC, num_subcores=NS
)


def _make_sc_agg(W):
    @functools.partial(
        pl.kernel,
        out_type=jax.ShapeDtypeStruct((NC, N, W), jnp.float32),
        mesh=_mesh,
        scratch_types=[
            pltpu.VMEM((IB, CH), jnp.int32),
            pltpu.VMEM((IB, CH), jnp.int32),
            pltpu.VMEM((CH, W), jnp.float32),
            pltpu.VMEM((CH, W), jnp.float32),
            pltpu.VMEM_SHARED((N, W), jnp.float32),
            pltpu.SemaphoreType.DMA,
            pltpu.SemaphoreType.DMA,
        ],
    )
    def sc_agg(src_hbm, dst_hbm, h_hbm, out_hbm, srcs_v, dsts_v, rows0, rows1,
               agg_sh, sem0, sem1):
        c = lax.axis_index("c")
        s = lax.axis_index("s")
        wid = c * NS + s
        rows = (rows0, rows1)
        sems = (sem0, sem1)

        def zbody(i, _):
            for k in range(W // 16):
                rows0[i, pl.ds(k * 16, 16)] = jnp.zeros((16,), jnp.float32)
            return 0

        lax.fori_loop(0, ZCH, zbody, 0)

        for j in range(ZPT):
            cid = s + j * NS

            @pl.when(cid < NZ)
            def _():
                pltpu.sync_copy(
                    rows0.at[pl.ds(0, ZCH)],
                    agg_sh.at[pl.ds(pl.multiple_of(cid * ZCH, ZCH), ZCH)],
                )

        plsc.subcore_barrier()

        def bbody(blk, _):
            b0 = pl.multiple_of(blk * IB, IB)
            pltpu.sync_copy(src_hbm.at[wid, pl.ds(b0, IB)], srcs_v)
            pltpu.sync_copy(dst_hbm.at[wid, pl.ds(b0, IB)], dsts_v)
            pltpu.async_copy(h_hbm.at[srcs_v.at[0]], rows0, sem0)
            pltpu.async_copy(h_hbm.at[srcs_v.at[1]], rows1, sem1)

            def ebody(g2, _):
                for b in range(2):
                    g = g2 * 2 + b
                    pltpu.make_async_copy(
                        h_hbm.at[srcs_v.at[g]], rows[b], sems[b]
                    ).wait()
                    pltpu.sync_copy(rows[b], agg_sh.at[dsts_v.at[g]], add=True)

                    @pl.when(g + 2 < IB)
                    def _():
                        pltpu.async_copy(
                            h_hbm.at[srcs_v.at[g + 2]], rows[b], sems[b]
                        )

                return 0

            lax.fori_loop(0, IB // 2, ebody, 0)
            return 0

        lax.fori_loop(0, NB, bbody, 0)
        plsc.subcore_barrier()

        for j in range(ZPT):
            cid = s + j * NS

            @pl.when(cid < NZ)
            def _():
                r0 = pl.multiple_of(cid * ZCH, ZCH)
                pltpu.sync_copy(agg_sh.at[pl.ds(r0, ZCH)], rows0.at[pl.ds(0, ZCH)])
                pltpu.sync_copy(rows0.at[pl.ds(0, ZCH)], out_hbm.at[c, pl.ds(r0, ZCH)])

    return sc_agg


_sc_agg = _make_sc_agg(H)


@functools.partial(
    pl.kernel,
    out_type=jax.ShapeDtypeStruct((NC, N), jnp.float32),
    mesh=_mesh,
    scratch_types=[
        pltpu.VMEM((NCH, CH), jnp.int32),
        pltpu.VMEM((CH,), jnp.float32),
        pltpu.VMEM((N,), jnp.float32),
        pltpu.VMEM_SHARED((N,), jnp.float32),
    ],
)
def _sc_deg(dst_hbm, out_hbm, dsts_v, ones_v, degw_v, deg_sh):
    c = lax.axis_index("c")
    s = lax.axis_index("s")
    wid = c * NS + s

    pltpu.sync_copy(dst_hbm.at[wid], dsts_v)

    for r0 in list(range(0, CH - 16, 16)) + [CH - 16]:
        ones_v[pl.ds(r0, 16)] = jnp.full((16,), 1.0, jnp.float32)

    def zbody(i, _):
        degw_v[pl.ds(i * 16, 16)] = jnp.zeros((16,), jnp.float32)
        return 0

    lax.fori_loop(0, N // 16, zbody, 0)

    @pl.when(s == 0)
    def _():
        pltpu.sync_copy(degw_v, deg_sh)

    plsc.subcore_barrier()

    def ebody(g, _):
        pltpu.sync_copy(ones_v, deg_sh.at[dsts_v.at[g]], add=True)
        return 0

    lax.fori_loop(0, NCH, ebody, 0)
    plsc.subcore_barrier()

    @pl.when(s == 0)
    def _():
        pltpu.sync_copy(deg_sh, degw_v)
        pltpu.sync_copy(degw_v, out_hbm.at[c])


RB = 1000


def _tc_init_body(x_ref, wi_ref, bi_ref, o_ref):
    o_ref[...] = jnp.maximum(
        jnp.dot(x_ref[...], wi_ref[...], preferred_element_type=jnp.float32)
        + bi_ref[...],
        0.0,
    )


_tc_init = pl.pallas_call(
    _tc_init_body,
    grid=(N // RB,),
    in_specs=[
        pl.BlockSpec((RB, D), lambda i: (i, 0)),
        pl.BlockSpec((D, H), lambda i: (0, 0)),
        pl.BlockSpec((1, H), lambda i: (0, 0)),
    ],
    out_specs=pl.BlockSpec((RB, H), lambda i: (i, 0)),
    out_shape=jax.ShapeDtypeStruct((N, H), jnp.float32),
)


def _tc_right_body(h_ref, wr_ref, bl_ref, o_ref):
    o_ref[...] = (
        jnp.dot(h_ref[...], wr_ref[...], preferred_element_type=jnp.float32)
        + bl_ref[...]
    )


_tc_right = pl.pallas_call(
    _tc_right_body,
    grid=(N // RB,),
    in_specs=[
        pl.BlockSpec((RB, H), lambda i: (i, 0)),
        pl.BlockSpec((H, H), lambda i: (0, 0)),
        pl.BlockSpec((1, H), lambda i: (0, 0)),
    ],
    out_specs=pl.BlockSpec((RB, H), lambda i: (i, 0)),
    out_shape=jax.ShapeDtypeStruct((N, H), jnp.float32),
)


def _norm_relu_res(agg_ref, deg_ref, r_ref, h_ref, wl_ref, g_ref, b_ref):
    d = jnp.clip(deg_ref[0] + deg_ref[1], 1.0, None)
    a = (agg_ref[0] + agg_ref[1]) / d
    h2 = jnp.dot(a, wl_ref[...], preferred_element_type=jnp.float32) + r_ref[...]
    mu = jnp.mean(h2, axis=-1, keepdims=True)
    var = jnp.mean((h2 - mu) ** 2, axis=-1, keepdims=True)
    h2 = (h2 - mu) * lax.rsqrt(var + 1e-5) * g_ref[...] + b_ref[...]
    return jnp.maximum(h2, 0.0) + h_ref[...]


def _tc_layer_body(agg_ref, deg_ref, r_ref, h_ref, wl_ref, g_ref, b_ref, o_ref):
    o_ref[...] = _norm_relu_res(agg_ref, deg_ref, r_ref, h_ref, wl_ref, g_ref,
                                b_ref)


def _tc_last_body(agg_ref, deg_ref, r_ref, h_ref, wl_ref, g_ref, b_ref,
                  wo_ref, bo_ref, o_ref):
    h = _norm_relu_res(agg_ref, deg_ref, r_ref, h_ref, wl_ref, g_ref, b_ref)
    logits = (
        jnp.dot(h, wo_ref[...], preferred_element_type=jnp.float32)
        + bo_ref[...]
    )
    m = jnp.max(logits, axis=-1, keepdims=True)
    lse = jnp.log(jnp.sum(jnp.exp(logits - m), axis=-1, keepdims=True)) + m
    o_ref[...] = logits - lse


_LAYER_SPECS = [
    pl.BlockSpec((NC, RB, H), lambda i: (0, i, 0)),
    pl.BlockSpec((NC, RB, 1), lambda i: (0, i, 0)),
    pl.BlockSpec((RB, H), lambda i: (i, 0)),
    pl.BlockSpec((RB, H), lambda i: (i, 0)),
    pl.BlockSpec((H, H), lambda i: (0, 0)),
    pl.BlockSpec((1, H), lambda i: (0, 0)),
    pl.BlockSpec((1, H), lambda i: (0, 0)),
]

_tc_layer = pl.pallas_call(
    _tc_layer_body,
    grid=(N // RB,),
    in_specs=_LAYER_SPECS,
    out_specs=pl.BlockSpec((RB, H), lambda i: (i, 0)),
    out_shape=jax.ShapeDtypeStruct((N, H), jnp.float32),
)

_tc_last = pl.pallas_call(
    _tc_last_body,
    grid=(N // RB,),
    in_specs=_LAYER_SPECS
    + [
        pl.BlockSpec((H, OUT), lambda i: (0, 0)),
        pl.BlockSpec((1, OUT), lambda i: (0, 0)),
    ],
    out_specs=pl.BlockSpec((RB, OUT), lambda i: (i, 0)),
    out_shape=jax.ShapeDtypeStruct((N, OUT), jnp.float32),
)


def kernel(x, edge_index, Wi, bi, Wl, bl, Wr, gamma, beta, Wo, bo):
    src = edge_index[0].reshape(NW, NCH, CH)
    dst = edge_index[1].reshape(NW, NCH, CH)
    deg2 = _sc_deg(dst)[:, :, None]
    h = _tc_init(x, Wi, bi[None, :])
    for i in range(LAYERS):
        agg2 = _sc_agg(src, dst, h)
        r = _tc_right(h, Wr[i], bl[i][None, :])
        args = (agg2, deg2, r, h, Wl[i], gamma[i][None, :], beta[i][None, :])
        if i < LAYERS - 1:
            h = _tc_layer(*args)
        else:
            return _tc_last(*args, Wo, bo[None, :])
